# baseline (device time: 502918 ns/iter reference)
import jax
import jax.numpy as jnp
from jax import lax
from jax.experimental import pallas as pl
from jax.experimental.pallas import tpu as pltpu

N_DEV = 16


def kernel(x, W1, W2):
    m, _ = x.shape
    d = W1.shape[1]
    f = W2.shape[1]
    blk = m // N_DEV

    def body(x_ref, w1_ref, w2_ref, out_ref, comm_ref, ag_ref,
             rs_send, rs_recv, ag_send, ag_recv, rs_credit, ag_credit):
        my = lax.axis_index("i")
        left = (my - 1) % N_DEV
        right = (my + 1) % N_DEV

        barrier = pltpu.get_barrier_semaphore()
        for nbr in (left, right):
            pl.semaphore_signal(barrier, inc=1, device_id=(nbr,),
                                device_id_type=pl.DeviceIdType.MESH)
        pl.semaphore_wait(barrier, 2)

        c0 = (my - 1) % N_DEV
        comm_ref[0, :, :] = jnp.dot(
            x_ref[pl.ds(c0 * blk, blk), :], w1_ref[:, :],
            preferred_element_type=jnp.float32)

        for s in range(N_DEV - 1):
            send_slot = s % 2
            recv_slot = (s + 1) % 2
            rdma = pltpu.make_async_remote_copy(
                src_ref=comm_ref.at[send_slot],
                dst_ref=comm_ref.at[recv_slot],
                send_sem=rs_send.at[send_slot],
                recv_sem=rs_recv.at[recv_slot],
                device_id=(right,),
                device_id_type=pl.DeviceIdType.MESH,
            )
            if s >= 1:
                pl.semaphore_wait(rs_credit, 1)
            rdma.start()
            j = (my - s - 2) % N_DEV
            p = jnp.dot(x_ref[pl.ds(j * blk, blk), :], w1_ref[:, :],
                        preferred_element_type=jnp.float32)
            rdma.wait_recv()
            comm_ref[recv_slot, :, :] = comm_ref[recv_slot, :, :] + p
            rdma.wait_send()
            if s <= N_DEV - 3:
                pl.semaphore_signal(rs_credit, inc=1, device_id=(left,),
                                    device_id_type=pl.DeviceIdType.MESH)

        h_slot = (N_DEV - 1) % 2
        for t in range(N_DEV - 1):
            recv_slot = (t + 1) % 2
            src = comm_ref.at[h_slot] if t == 0 else ag_ref.at[t % 2]
            rdma = pltpu.make_async_remote_copy(
                src_ref=src,
                dst_ref=ag_ref.at[recv_slot],
                send_sem=ag_send.at[t % 2],
                recv_sem=ag_recv.at[recv_slot],
                device_id=(right,),
                device_id_type=pl.DeviceIdType.MESH,
            )
            if t >= 2:
                pl.semaphore_wait(ag_credit, 1)
            rdma.start()
            if t == 0:
                out_ref[pl.ds(my * blk, blk), :] = jnp.dot(
                    comm_ref[h_slot, :, :], w2_ref[:, :],
                    preferred_element_type=jnp.float32)
            rdma.wait_recv()
            j = (my - 1 - t) % N_DEV
            out_ref[pl.ds(j * blk, blk), :] = jnp.dot(
                ag_ref[recv_slot, :, :], w2_ref[:, :],
                preferred_element_type=jnp.float32)
            rdma.wait_send()
            if 1 <= t <= N_DEV - 3:
                pl.semaphore_signal(ag_credit, inc=1, device_id=(left,),
                                    device_id_type=pl.DeviceIdType.MESH)

    return pl.pallas_call(
        body,
        out_shape=jax.ShapeDtypeStruct((m, f), jnp.float32),
        in_specs=[
            pl.BlockSpec(memory_space=pltpu.VMEM),
            pl.BlockSpec(memory_space=pltpu.VMEM),
            pl.BlockSpec(memory_space=pltpu.VMEM),
        ],
        out_specs=pl.BlockSpec(memory_space=pltpu.VMEM),
        scratch_shapes=[
            pltpu.VMEM((2, blk, d), jnp.float32),
            pltpu.VMEM((2, blk, d), jnp.float32),
            pltpu.SemaphoreType.DMA((2,)),
            pltpu.SemaphoreType.DMA((2,)),
            pltpu.SemaphoreType.DMA((2,)),
            pltpu.SemaphoreType.DMA((2,)),
            pltpu.SemaphoreType.REGULAR,
            pltpu.SemaphoreType.REGULAR,
        ],
        compiler_params=pltpu.CompilerParams(collective_id=0),
    )(x, W1, W2)


# device time: 339313 ns/iter; 1.4822x vs baseline; 1.4822x over previous
import jax
import jax.numpy as jnp
from jax import lax
from jax.experimental import pallas as pl
from jax.experimental.pallas import tpu as pltpu

N_DEV = 16


def kernel(x, W1, W2):
    m, _ = x.shape
    d = W1.shape[1]
    f = W2.shape[1]
    blk = m // N_DEV
    dh = d // 2

    def body(x_ref, w1_ref, w2_ref, out_ref,
             commA, commB, agA, agB,
             rsA_send, rsA_recv, rsB_send, rsB_recv,
             agA_send, agA_recv, agB_send, agB_recv,
             rsA_credit, rsB_credit, agA_credit, agB_credit):
        my = lax.axis_index("i")
        left = (my - 1) % N_DEV
        right = (my + 1) % N_DEV

        barrier = pltpu.get_barrier_semaphore()
        for nbr in (left, right):
            pl.semaphore_signal(barrier, inc=1, device_id=(nbr,),
                                device_id_type=pl.DeviceIdType.MESH)
        pl.semaphore_wait(barrier, 2)

        cA0 = (my - 1) % N_DEV
        cB0 = (my + 1) % N_DEV
        commA[0, :, :] = jnp.dot(
            x_ref[pl.ds(cA0 * blk, blk), :], w1_ref[:, :dh],
            preferred_element_type=jnp.float32)
        commB[0, :, :] = jnp.dot(
            x_ref[pl.ds(cB0 * blk, blk), :], w1_ref[:, dh:],
            preferred_element_type=jnp.float32)

        for s in range(N_DEV - 1):
            send_slot = s % 2
            recv_slot = (s + 1) % 2
            rdmaA = pltpu.make_async_remote_copy(
                src_ref=commA.at[send_slot], dst_ref=commA.at[recv_slot],
                send_sem=rsA_send.at[send_slot], recv_sem=rsA_recv.at[recv_slot],
                device_id=(right,), device_id_type=pl.DeviceIdType.MESH)
            rdmaB = pltpu.make_async_remote_copy(
                src_ref=commB.at[send_slot], dst_ref=commB.at[recv_slot],
                send_sem=rsB_send.at[send_slot], recv_sem=rsB_recv.at[recv_slot],
                device_id=(left,), device_id_type=pl.DeviceIdType.MESH)
            if s >= 1:
                pl.semaphore_wait(rsA_credit, 1)
                pl.semaphore_wait(rsB_credit, 1)
            rdmaA.start()
            rdmaB.start()
            jA = (my - s - 2) % N_DEV
            jB = (my + s + 2) % N_DEV
            pA = jnp.dot(x_ref[pl.ds(jA * blk, blk), :], w1_ref[:, :dh],
                         preferred_element_type=jnp.float32)
            pB = jnp.dot(x_ref[pl.ds(jB * blk, blk), :], w1_ref[:, dh:],
                         preferred_element_type=jnp.float32)
            rdmaA.wait_recv()
            commA[recv_slot, :, :] = commA[recv_slot, :, :] + pA
            rdmaB.wait_recv()
            commB[recv_slot, :, :] = commB[recv_slot, :, :] + pB
            rdmaA.wait_send()
            rdmaB.wait_send()
            if s <= N_DEV - 3:
                pl.semaphore_signal(rsA_credit, inc=1, device_id=(left,),
                                    device_id_type=pl.DeviceIdType.MESH)
                pl.semaphore_signal(rsB_credit, inc=1, device_id=(right,),
                                    device_id_type=pl.DeviceIdType.MESH)

        h_slot = (N_DEV - 1) % 2
        for t in range(N_DEV - 1):
            recv_slot = (t + 1) % 2
            srcA = commA.at[h_slot] if t == 0 else agA.at[t % 2]
            srcB = commB.at[h_slot] if t == 0 else agB.at[t % 2]
            rdmaA = pltpu.make_async_remote_copy(
                src_ref=srcA, dst_ref=agA.at[recv_slot],
                send_sem=agA_send.at[t % 2], recv_sem=agA_recv.at[recv_slot],
                device_id=(right,), device_id_type=pl.DeviceIdType.MESH)
            rdmaB = pltpu.make_async_remote_copy(
                src_ref=srcB, dst_ref=agB.at[recv_slot],
                send_sem=agB_send.at[t % 2], recv_sem=agB_recv.at[recv_slot],
                device_id=(left,), device_id_type=pl.DeviceIdType.MESH)
            if t >= 2:
                pl.semaphore_wait(agA_credit, 1)
                pl.semaphore_wait(agB_credit, 1)
            rdmaA.start()
            rdmaB.start()
            if t == 0:
                out_ref[pl.ds(my * blk, blk), :] = (
                    jnp.dot(commA[h_slot, :, :], w2_ref[:dh, :],
                            preferred_element_type=jnp.float32)
                    + jnp.dot(commB[h_slot, :, :], w2_ref[dh:, :],
                              preferred_element_type=jnp.float32))
            rdmaA.wait_recv()
            rdmaB.wait_recv()
            jA = (my - 1 - t) % N_DEV
            jB = (my + 1 + t) % N_DEV
            rA = jnp.dot(agA[recv_slot, :, :], w2_ref[:dh, :],
                         preferred_element_type=jnp.float32)
            rB = jnp.dot(agB[recv_slot, :, :], w2_ref[dh:, :],
                         preferred_element_type=jnp.float32)
            half = (N_DEV - 2) // 2
            if t < half:
                out_ref[pl.ds(jA * blk, blk), :] = rA
                out_ref[pl.ds(jB * blk, blk), :] = rB
            elif t == half:
                out_ref[pl.ds(jA * blk, blk), :] = rA + rB
            else:
                out_ref[pl.ds(jA * blk, blk), :] = (
                    out_ref[pl.ds(jA * blk, blk), :] + rA)
                out_ref[pl.ds(jB * blk, blk), :] = (
                    out_ref[pl.ds(jB * blk, blk), :] + rB)
            rdmaA.wait_send()
            rdmaB.wait_send()
            if 1 <= t <= N_DEV - 3:
                pl.semaphore_signal(agA_credit, inc=1, device_id=(left,),
                                    device_id_type=pl.DeviceIdType.MESH)
                pl.semaphore_signal(agB_credit, inc=1, device_id=(right,),
                                    device_id_type=pl.DeviceIdType.MESH)

    return pl.pallas_call(
        body,
        out_shape=jax.ShapeDtypeStruct((m, f), jnp.float32),
        in_specs=[
            pl.BlockSpec(memory_space=pltpu.VMEM),
            pl.BlockSpec(memory_space=pltpu.VMEM),
            pl.BlockSpec(memory_space=pltpu.VMEM),
        ],
        out_specs=pl.BlockSpec(memory_space=pltpu.VMEM),
        scratch_shapes=[
            pltpu.VMEM((2, blk, dh), jnp.float32),
            pltpu.VMEM((2, blk, dh), jnp.float32),
            pltpu.VMEM((2, blk, dh), jnp.float32),
            pltpu.VMEM((2, blk, dh), jnp.float32),
            pltpu.SemaphoreType.DMA((2,)),
            pltpu.SemaphoreType.DMA((2,)),
            pltpu.SemaphoreType.DMA((2,)),
            pltpu.SemaphoreType.DMA((2,)),
            pltpu.SemaphoreType.DMA((2,)),
            pltpu.SemaphoreType.DMA((2,)),
            pltpu.SemaphoreType.DMA((2,)),
            pltpu.SemaphoreType.DMA((2,)),
            pltpu.SemaphoreType.REGULAR,
            pltpu.SemaphoreType.REGULAR,
            pltpu.SemaphoreType.REGULAR,
            pltpu.SemaphoreType.REGULAR,
        ],
        compiler_params=pltpu.CompilerParams(collective_id=0),
    )(x, W1, W2)


# device time: 190620 ns/iter; 2.6383x vs baseline; 1.7800x over previous
import jax
import jax.numpy as jnp
from jax import lax
from jax.experimental import pallas as pl
from jax.experimental.pallas import tpu as pltpu

N_DEV = 16
NSLOT = 4
S = 2


def kernel(x, W1, W2):
    m, _ = x.shape
    d = W1.shape[1]
    f = W2.shape[1]
    blk = m // N_DEV
    dh = d // 2
    qr = blk // S

    def body(x_ref, w1_ref, w2_ref, out_ref,
             commA, commB, agA, agB,
             rsA_send, rsA_recv, rsB_send, rsB_recv,
             agA_send, agA_recv, agB_send, agB_recv,
             rsA_credit, rsB_credit, agA_credit, agB_credit):
        my = lax.axis_index("i")
        left = (my - 1) % N_DEV
        right = (my + 1) % N_DEV

        def psub(block, q, lo_half):
            w = w1_ref[:, :dh] if lo_half else w1_ref[:, dh:]
            return jnp.dot(
                x_ref[pl.ds(block * blk + q * qr, qr), :], w,
                preferred_element_type=jnp.float32)

        def mk(buf, slot_src, slot_dst, q, ssem, rsem, dev):
            return pltpu.make_async_remote_copy(
                src_ref=buf.at[slot_src, q],
                dst_ref=buf.at[slot_dst, q],
                send_sem=ssem.at[slot_dst, q],
                recv_sem=rsem.at[slot_dst, q],
                device_id=(dev,),
                device_id_type=pl.DeviceIdType.MESH)

        cA0 = (my - 1) % N_DEV
        cB0 = (my + 1) % N_DEV
        for q in range(S):
            commA[NSLOT - 1, q] = psub(cA0, q, True)
            commB[NSLOT - 1, q] = psub(cB0, q, False)

        barrier = pltpu.get_barrier_semaphore()
        for nbr in (left, right):
            pl.semaphore_signal(barrier, inc=1, device_id=(nbr,),
                                device_id_type=pl.DeviceIdType.MESH)
        pl.semaphore_wait(barrier, 2)

        dA, dB = {}, {}
        for q in range(S):
            dA[(0, q)] = mk(commA, NSLOT - 1, 0, q, rsA_send, rsA_recv, right)
            dB[(0, q)] = mk(commB, NSLOT - 1, 0, q, rsB_send, rsB_recv, left)
            dA[(0, q)].start()
            dB[(0, q)].start()

        for s in range(N_DEV - 1):
            slot = s % NSLOT
            jA = (my - s - 2) % N_DEV
            jB = (my + s + 2) % N_DEV
            for q in range(S):
                pA = psub(jA, q, True)
                pB = psub(jB, q, False)
                dA[(s, q)].wait_recv()
                commA[slot, q] = commA[slot, q] + pA
                if s < N_DEV - 2:
                    h = s + 1
                    if h >= NSLOT - 1 and q == 0:
                        pl.semaphore_wait(rsA_credit, 1)
                    dA[(h, q)] = mk(commA, slot, h % NSLOT, q,
                                    rsA_send, rsA_recv, right)
                    dA[(h, q)].start()
                dB[(s, q)].wait_recv()
                commB[slot, q] = commB[slot, q] + pB
                if s < N_DEV - 2:
                    h = s + 1
                    if h >= NSLOT - 1 and q == 0:
                        pl.semaphore_wait(rsB_credit, 1)
                    dB[(h, q)] = mk(commB, slot, h % NSLOT, q,
                                    rsB_send, rsB_recv, left)
                    dB[(h, q)].start()
            for q in range(S):
                dA[(s, q)].wait_send()
                dB[(s, q)].wait_send()
            if s <= N_DEV - 5:
                pl.semaphore_signal(rsA_credit, inc=1, device_id=(left,),
                                    device_id_type=pl.DeviceIdType.MESH)
                pl.semaphore_signal(rsB_credit, inc=1, device_id=(right,),
                                    device_id_type=pl.DeviceIdType.MESH)

        h_slot = (N_DEV - 2) % NSLOT

        gA, gB = {}, {}
        for q in range(S):
            gA[(0, q)] = mk2 = pltpu.make_async_remote_copy(
                src_ref=commA.at[h_slot, q], dst_ref=agA.at[0, q],
                send_sem=agA_send.at[0, q], recv_sem=agA_recv.at[0, q],
                device_id=(right,), device_id_type=pl.DeviceIdType.MESH)
            gB[(0, q)] = mk3 = pltpu.make_async_remote_copy(
                src_ref=commB.at[h_slot, q], dst_ref=agB.at[0, q],
                send_sem=agB_send.at[0, q], recv_sem=agB_recv.at[0, q],
                device_id=(left,), device_id_type=pl.DeviceIdType.MESH)
            mk2.start()
            mk3.start()

        def gemm2_store(t):
            if t == 0:
                for q in range(S):
                    r = (jnp.dot(commA[h_slot, q], w2_ref[:dh, :],
                                 preferred_element_type=jnp.float32)
                         + jnp.dot(commB[h_slot, q], w2_ref[dh:, :],
                                   preferred_element_type=jnp.float32))
                    out_ref[pl.ds(my * blk + q * qr, qr), :] = r
                return
            slot = (t - 1) % NSLOT
            jA = (my - t) % N_DEV
            jB = (my + t) % N_DEV
            for q in range(S):
                rA = jnp.dot(agA[slot, q], w2_ref[:dh, :],
                             preferred_element_type=jnp.float32)
                rB = jnp.dot(agB[slot, q], w2_ref[dh:, :],
                             preferred_element_type=jnp.float32)
                rowA = pl.ds(jA * blk + q * qr, qr)
                rowB = pl.ds(jB * blk + q * qr, qr)
                if t < N_DEV // 2:
                    out_ref[rowA, :] = rA
                    out_ref[rowB, :] = rB
                elif t == N_DEV // 2:
                    out_ref[rowA, :] = rA + rB
                else:
                    out_ref[rowA, :] = out_ref[rowA, :] + rA
                    out_ref[rowB, :] = out_ref[rowB, :] + rB

        for t in range(N_DEV - 1):
            slot = t % NSLOT
            for q in range(S):
                gA[(t, q)].wait_recv()
                if t < N_DEV - 2:
                    h = t + 1
                    if h >= NSLOT and q == 0:
                        pl.semaphore_wait(agA_credit, 1)
                    gA[(h, q)] = mk(agA, slot, h % NSLOT, q,
                                    agA_send, agA_recv, right)
                    gA[(h, q)].start()
                gB[(t, q)].wait_recv()
                if t < N_DEV - 2:
                    h = t + 1
                    if h >= NSLOT and q == 0:
                        pl.semaphore_wait(agB_credit, 1)
                    gB[(h, q)] = mk(agB, slot, h % NSLOT, q,
                                    agB_send, agB_recv, left)
                    gB[(h, q)].start()
            gemm2_store(t)
            for q in range(S):
                gA[(t, q)].wait_send()
                gB[(t, q)].wait_send()
            if 1 <= t <= N_DEV - 5:
                pl.semaphore_signal(agA_credit, inc=1, device_id=(left,),
                                    device_id_type=pl.DeviceIdType.MESH)
                pl.semaphore_signal(agB_credit, inc=1, device_id=(right,),
                                    device_id_type=pl.DeviceIdType.MESH)
        gemm2_store(N_DEV - 1)

    return pl.pallas_call(
        body,
        out_shape=jax.ShapeDtypeStruct((m, f), jnp.float32),
        in_specs=[
            pl.BlockSpec(memory_space=pltpu.VMEM),
            pl.BlockSpec(memory_space=pltpu.VMEM),
            pl.BlockSpec(memory_space=pltpu.VMEM),
        ],
        out_specs=pl.BlockSpec(memory_space=pltpu.VMEM),
        scratch_shapes=[
            pltpu.VMEM((NSLOT, S, qr, dh), jnp.float32),
            pltpu.VMEM((NSLOT, S, qr, dh), jnp.float32),
            pltpu.VMEM((NSLOT, S, qr, dh), jnp.float32),
            pltpu.VMEM((NSLOT, S, qr, dh), jnp.float32),
            pltpu.SemaphoreType.DMA((NSLOT, S)),
            pltpu.SemaphoreType.DMA((NSLOT, S)),
            pltpu.SemaphoreType.DMA((NSLOT, S)),
            pltpu.SemaphoreType.DMA((NSLOT, S)),
            pltpu.SemaphoreType.DMA((NSLOT, S)),
            pltpu.SemaphoreType.DMA((NSLOT, S)),
            pltpu.SemaphoreType.DMA((NSLOT, S)),
            pltpu.SemaphoreType.DMA((NSLOT, S)),
            pltpu.SemaphoreType.REGULAR,
            pltpu.SemaphoreType.REGULAR,
            pltpu.SemaphoreType.REGULAR,
            pltpu.SemaphoreType.REGULAR,
        ],
        compiler_params=pltpu.CompilerParams(collective_id=0),
    )(x, W1, W2)
